# baseline (speedup 1.0000x reference)
import functools

import jax
import jax.numpy as jnp
from jax import lax
from jax.experimental import pallas as pl
from jax.experimental.pallas import tpu as pltpu
from jax.experimental.pallas import tpu_sc as plsc

B, T, C = 1, 2048, 768
E, K, F = 8, 2, 2048
N = T * K
BLK = 128
NB = (N + E * (BLK - 1)) // BLK + 1
CAP = NB * BLK
NC, NS, L = 2, 16, 16
NW = NC * NS
APW = N // NW
TPW = T // NW
NBX = 48

_sc_mesh = plsc.VectorSubcoreMesh(
    core_axis_name="c", subcore_axis_name="s", num_cores=NC, num_subcores=NS)
_sc_params = pltpu.CompilerParams(needs_layout_passes=False)


def _wid():
    return lax.axis_index("s") * NC + lax.axis_index("c")


def _router_body(x_ref, wr_ref, eidx_ref, wgt_ref, aux_ref):
    x = x_ref[...]
    logits = jnp.dot(x, wr_ref[...], preferred_element_type=jnp.float32)
    m = jnp.max(logits, axis=-1, keepdims=True)
    ex = jnp.exp(logits - m)
    probs = ex / jnp.sum(ex, axis=-1, keepdims=True)
    i0 = jnp.argmax(probs, axis=-1).astype(jnp.int32)
    p0 = jnp.max(probs, axis=-1)
    cols = lax.broadcasted_iota(jnp.int32, probs.shape, 1)
    masked = jnp.where(cols == i0[:, None], -jnp.inf, probs)
    i1 = jnp.argmax(masked, axis=-1).astype(jnp.int32)
    p1 = jnp.max(masked, axis=-1)
    den = p0 + p1
    eidx_ref[...] = jnp.concatenate([i0[None, :], i1[None, :]], axis=0)
    wgt_ref[...] = jnp.concatenate([(p0 / den)[None, :], (p1 / den)[None, :]],
                                   axis=0)
    avg = jnp.sum(probs, axis=0, keepdims=True) / T
    aux_ref[...] = E * jnp.sum(avg * avg, keepdims=True)


def _router(x2, w_router):
    return pl.pallas_call(
        _router_body,
        out_shape=[
            jax.ShapeDtypeStruct((K, T), jnp.int32),
            jax.ShapeDtypeStruct((K, T), jnp.float32),
            jax.ShapeDtypeStruct((1, 1), jnp.float32),
        ],
    )(x2, w_router)


@functools.partial(
    pl.kernel,
    out_type=[
        jax.ShapeDtypeStruct((N,), jnp.int32),
        jax.ShapeDtypeStruct((CAP, C // 2), jnp.int32),
        jax.ShapeDtypeStruct((NBX,), jnp.int32),
    ],
    mesh=_sc_mesh,
    scratch_types=[
        pltpu.VMEM((N,), jnp.int32),
        pltpu.VMEM((APW,), jnp.int32),
        pltpu.VMEM((APW, C // 2), jnp.int32),
        pltpu.VMEM((NBX,), jnp.int32),
        pltpu.SemaphoreType.DMA,
    ],
    compiler_params=_sc_params,
)
def _route(eidx_hbm, x_hbm, pos_hbm, xs_hbm, bexp_hbm, ev, posb, rows, bexpb,
           sem):
    wid = _wid()
    base = wid * APW
    t0 = base - jnp.where(wid >= NW // K, T, 0)
    cp = pltpu.async_copy(x_hbm.at[pl.ds(t0, APW)], rows, sem)
    pltpu.sync_copy(eidx_hbm, ev)
    my_vreg = wid * (APW // L)

    def hist_step(i, carry):
        cntv = carry[:E]
        snapv = carry[E:]
        snapv = tuple(
            jnp.where(i == my_vreg, cntv[e], snapv[e]) for e in range(E))
        v = ev[pl.ds(i * L, L)]
        cntv = tuple(
            cntv[e] + jnp.where(v == e, 1, 0).astype(jnp.int32)
            for e in range(E))
        return cntv + snapv

    zv = jnp.zeros((L,), jnp.int32)
    carry = lax.fori_loop(0, N // L, hist_step, (zv,) * (2 * E))
    tot = [jnp.sum(carry[e]) for e in range(E)]
    pre = [jnp.sum(carry[E + e]) for e in range(E)]

    run = jnp.int32(0)
    start, sb = [], []
    for e in range(E):
        start.append(run)
        sb.append(run // BLK)
        padded = ((tot[e] + BLK - 1) // BLK) * BLK
        run = run + padded
    nb_used = run // BLK
    mybase = [start[e] + pre[e] for e in range(E)]

    for r in range(APW // L):
        v = ev[pl.ds(base + r * L, L)]
        posv = jnp.zeros((L,), jnp.int32)
        for e in range(E):
            m = v == e
            mi = jnp.where(m, 1, 0).astype(jnp.int32)
            c = plsc.cumsum(mi)
            posv = jnp.where(m, mybase[e] + c - 1, posv)
            mybase[e] = mybase[e] + jnp.sum(mi)
        posb[pl.ds(r * L, L)] = posv

    pltpu.sync_copy(posb, pos_hbm.at[pl.ds(base, APW)])
    cp.wait()
    pltpu.async_copy(rows, xs_hbm.at[posb], sem).wait()

    @pl.when(wid == 0)
    def _():
        for ch in range(NBX // L):
            bv = jnp.int32(ch * L) + lax.iota(jnp.int32, L)
            val = jnp.zeros((L,), jnp.int32)
            for e in range(1, E):
                val = val + jnp.where(bv >= sb[e], 1, 0).astype(jnp.int32)
            val = jnp.where(bv == NB, nb_used, val)
            bexpb[pl.ds(ch * L, L)] = val
        pltpu.sync_copy(bexpb, bexp_hbm)


def _ffn_body(bexp_ref, xs_ref, wv_ref, wp_ref, ys_ref):
    b = pl.program_id(0)

    @pl.when(b < bexp_ref[NB])
    def _():
        xb = xs_ref[...]
        gv = jnp.dot(xb, wv_ref[0], preferred_element_type=jnp.float32)
        g = gv[:, :F]
        v = gv[:, F:]
        h = (g * lax.logistic(g) * v).astype(jnp.bfloat16)
        ys_ref[...] = jnp.dot(h, wp_ref[0], preferred_element_type=jnp.float32)


def _ffn(bexp, xs_bf, wv_b, wp_b):
    grid_spec = pltpu.PrefetchScalarGridSpec(
        num_scalar_prefetch=1,
        grid=(NB,),
        in_specs=[
            pl.BlockSpec((BLK, C), lambda b, bexp: (b, 0)),
            pl.BlockSpec((1, C, 2 * F), lambda b, bexp: (bexp[b], 0, 0)),
            pl.BlockSpec((1, F, C), lambda b, bexp: (bexp[b], 0, 0)),
        ],
        out_specs=pl.BlockSpec((BLK, C), lambda b, bexp: (b, 0)),
    )
    return pl.pallas_call(
        _ffn_body,
        grid_spec=grid_spec,
        out_shape=jax.ShapeDtypeStruct((CAP, C), jnp.float32),
        compiler_params=pltpu.CompilerParams(
            dimension_semantics=("arbitrary",),
            vmem_limit_bytes=100 * 1024 * 1024),
    )(bexp, xs_bf, wv_b, wp_b)


_HALF = TPW // 2


@functools.partial(
    pl.kernel,
    out_type=jax.ShapeDtypeStruct((T, C), jnp.float32),
    mesh=_sc_mesh,
    scratch_types=[
        pltpu.VMEM((_HALF,), jnp.int32),
        pltpu.VMEM((_HALF,), jnp.int32),
        pltpu.VMEM((_HALF + L,), jnp.float32),
        pltpu.VMEM((_HALF + L,), jnp.float32),
        pltpu.VMEM((_HALF, C), jnp.float32),
        pltpu.VMEM((_HALF, C), jnp.float32),
        pltpu.VMEM((_HALF, C), jnp.float32),
        pltpu.SemaphoreType.DMA,
        pltpu.SemaphoreType.DMA,
    ],
    compiler_params=_sc_params,
)
def _combine(ys_hbm, pos_hbm, wgt_hbm, out_hbm, p0v, p1v, w0v, w1v, buf0,
             buf1, obuf, sem0, sem1):
    tbase = _wid() * TPW
    for h in range(2):
        tb = tbase + h * _HALF
        pltpu.sync_copy(pos_hbm.at[pl.ds(tb, _HALF)], p0v)
        pltpu.sync_copy(pos_hbm.at[pl.ds(T + tb, _HALF)], p1v)
        pltpu.sync_copy(wgt_hbm.at[pl.ds(tb, _HALF)],
                        w0v.at[pl.ds(0, _HALF)])
        pltpu.sync_copy(wgt_hbm.at[pl.ds(T + tb, _HALF)],
                        w1v.at[pl.ds(0, _HALF)])
        c0 = pltpu.async_copy(ys_hbm.at[p0v], buf0, sem0)
        c1 = pltpu.async_copy(ys_hbm.at[p1v], buf1, sem1)
        c0.wait()
        c1.wait()

        def tok(i, _):
            w0 = w0v[pl.ds(i, L)][0]
            w1 = w1v[pl.ds(i, L)][0]
            for k2 in range(C // L):
                s = pl.ds(k2 * L, L)
                obuf[i, s] = w0 * buf0[i, s] + w1 * buf1[i, s]
            return 0

        lax.fori_loop(0, _HALF, tok, 0)
        pltpu.sync_copy(obuf, out_hbm.at[pl.ds(tb, _HALF)])


def kernel(x, w_router, w_v, w_proj):
    x2 = x.reshape(T, C)
    x_pack = lax.bitcast_convert_type(
        x2.astype(jnp.bfloat16).reshape(T, C // 2, 2), jnp.int32)
    wv_b = w_v.astype(jnp.bfloat16)
    wp_b = w_proj.astype(jnp.bfloat16)
    eidx, wgt, aux = _router(x2, w_router)
    pos, xs_pack, bexp = _route(eidx.reshape(N), x_pack)
    xs_bf = lax.bitcast_convert_type(xs_pack, jnp.bfloat16).reshape(CAP, C)
    ys = _ffn(bexp, xs_bf, wv_b, wp_b)
    out2 = _combine(ys, pos, wgt.reshape(N))
    return out2.reshape(B, T, C), aux.reshape(())


# speedup vs baseline: 1.5588x; 1.5588x over previous
import functools

import jax
import jax.numpy as jnp
from jax import lax
from jax.experimental import pallas as pl
from jax.experimental.pallas import tpu as pltpu
from jax.experimental.pallas import tpu_sc as plsc

B, T, C = 1, 2048, 768
E, K, F = 8, 2, 2048
N = T * K
BLK = 128
NB = (N + E * (BLK - 1)) // BLK + 1
CAP = NB * BLK
NC, NS, L = 2, 16, 16
NW = NC * NS
APW = N // NW
TPW = T // NW
NBX = 48

_sc_mesh = plsc.VectorSubcoreMesh(
    core_axis_name="c", subcore_axis_name="s", num_cores=NC, num_subcores=NS)
_sc_params = pltpu.CompilerParams(needs_layout_passes=False)


def _wid():
    return lax.axis_index("s") * NC + lax.axis_index("c")


def _router_body(x_ref, wr_ref, eidx_ref, wgt_ref, aux_ref):
    x = x_ref[...]
    logits = jnp.dot(x, wr_ref[...], preferred_element_type=jnp.float32)
    m = jnp.max(logits, axis=-1, keepdims=True)
    ex = jnp.exp(logits - m)
    probs = ex / jnp.sum(ex, axis=-1, keepdims=True)
    i0 = jnp.argmax(probs, axis=-1).astype(jnp.int32)
    p0 = jnp.max(probs, axis=-1)
    cols = lax.broadcasted_iota(jnp.int32, probs.shape, 1)
    masked = jnp.where(cols == i0[:, None], -jnp.inf, probs)
    i1 = jnp.argmax(masked, axis=-1).astype(jnp.int32)
    p1 = jnp.max(masked, axis=-1)
    den = p0 + p1
    eidx_ref[...] = jnp.concatenate([i0[None, :], i1[None, :]], axis=0)
    wgt_ref[...] = jnp.concatenate([(p0 / den)[None, :], (p1 / den)[None, :]],
                                   axis=0)
    avg = jnp.sum(probs, axis=0, keepdims=True) / T
    aux_ref[...] = E * jnp.sum(avg * avg, keepdims=True)


def _router(x2, w_router):
    return pl.pallas_call(
        _router_body,
        out_shape=[
            jax.ShapeDtypeStruct((K, T), jnp.int32),
            jax.ShapeDtypeStruct((K, T), jnp.float32),
            jax.ShapeDtypeStruct((1, 1), jnp.float32),
        ],
    )(x2, w_router)


@functools.partial(
    pl.kernel,
    out_type=[
        jax.ShapeDtypeStruct((N,), jnp.int32),
        jax.ShapeDtypeStruct((CAP, C), jnp.float32),
        jax.ShapeDtypeStruct((NBX,), jnp.int32),
    ],
    mesh=_sc_mesh,
    scratch_types=[
        pltpu.VMEM((N,), jnp.int32),
        pltpu.VMEM((APW,), jnp.int32),
        pltpu.VMEM((APW, C), jnp.float32),
        pltpu.VMEM((NBX,), jnp.int32),
        pltpu.SemaphoreType.DMA,
    ],
    compiler_params=_sc_params,
)
def _route(eidx_hbm, x_hbm, pos_hbm, xs_hbm, bexp_hbm, ev, posb, rows, bexpb,
           sem):
    wid = _wid()
    base = wid * APW
    t0 = base - jnp.where(wid >= NW // K, T, 0)
    cp = pltpu.async_copy(x_hbm.at[pl.ds(t0, APW)], rows, sem)
    pltpu.sync_copy(eidx_hbm, ev)
    my_vreg = wid * (APW // L)

    def hist_step(i, carry):
        cntv = carry[:E]
        snapv = carry[E:]
        snapv = tuple(
            jnp.where(i == my_vreg, cntv[e], snapv[e]) for e in range(E))
        v = ev[pl.ds(i * L, L)]
        cntv = tuple(
            cntv[e] + jnp.where(v == e, 1, 0).astype(jnp.int32)
            for e in range(E))
        return cntv + snapv

    zv = jnp.zeros((L,), jnp.int32)
    carry = lax.fori_loop(0, N // L, hist_step, (zv,) * (2 * E))
    tot = [jnp.sum(carry[e]) for e in range(E)]
    pre = [jnp.sum(carry[E + e]) for e in range(E)]

    run = jnp.int32(0)
    start, sb = [], []
    for e in range(E):
        start.append(run)
        sb.append(run // BLK)
        padded = ((tot[e] + BLK - 1) // BLK) * BLK
        run = run + padded
    nb_used = run // BLK
    mybase = [start[e] + pre[e] for e in range(E)]

    for r in range(APW // L):
        v = ev[pl.ds(base + r * L, L)]
        posv = jnp.zeros((L,), jnp.int32)
        for e in range(E):
            m = v == e
            mi = jnp.where(m, 1, 0).astype(jnp.int32)
            c = plsc.cumsum(mi)
            posv = jnp.where(m, mybase[e] + c - 1, posv)
            mybase[e] = mybase[e] + jnp.sum(mi)
        posb[pl.ds(r * L, L)] = posv

    pltpu.sync_copy(posb, pos_hbm.at[pl.ds(base, APW)])
    cp.wait()
    pltpu.async_copy(rows, xs_hbm.at[posb], sem).wait()

    @pl.when(wid == 0)
    def _():
        for ch in range(NBX // L):
            bv = jnp.int32(ch * L) + lax.iota(jnp.int32, L)
            val = jnp.zeros((L,), jnp.int32)
            for e in range(1, E):
                val = val + jnp.where(bv >= sb[e], 1, 0).astype(jnp.int32)
            val = jnp.where(bv == NB, nb_used, val)
            bexpb[pl.ds(ch * L, L)] = val
        pltpu.sync_copy(bexpb, bexp_hbm)


def _ffn_body(bexp_ref, xs_ref, wv_ref, wp_ref, ys_ref):
    b = pl.program_id(0)

    @pl.when(b < bexp_ref[NB])
    def _():
        xb = xs_ref[...].astype(jnp.bfloat16)
        gv = jnp.dot(xb, wv_ref[0], preferred_element_type=jnp.float32)
        g = gv[:, :F]
        v = gv[:, F:]
        h = (g * lax.logistic(g) * v).astype(jnp.bfloat16)
        ys_ref[...] = jnp.dot(h, wp_ref[0], preferred_element_type=jnp.float32)


def _ffn(bexp, xs_bf, wv_b, wp_b):
    grid_spec = pltpu.PrefetchScalarGridSpec(
        num_scalar_prefetch=1,
        grid=(NB,),
        in_specs=[
            pl.BlockSpec((BLK, C), lambda b, bexp: (b, 0)),
            pl.BlockSpec((1, C, 2 * F), lambda b, bexp: (bexp[b], 0, 0)),
            pl.BlockSpec((1, F, C), lambda b, bexp: (bexp[b], 0, 0)),
        ],
        out_specs=pl.BlockSpec((BLK, C), lambda b, bexp: (b, 0)),
    )
    return pl.pallas_call(
        _ffn_body,
        grid_spec=grid_spec,
        out_shape=jax.ShapeDtypeStruct((CAP, C), jnp.float32),
        compiler_params=pltpu.CompilerParams(
            dimension_semantics=("arbitrary",)),
    )(bexp, xs_bf, wv_b, wp_b)


_HALF = TPW // 2


@functools.partial(
    pl.kernel,
    out_type=jax.ShapeDtypeStruct((T, C), jnp.float32),
    mesh=_sc_mesh,
    scratch_types=[
        pltpu.VMEM((_HALF,), jnp.int32),
        pltpu.VMEM((_HALF,), jnp.int32),
        pltpu.VMEM((_HALF + L,), jnp.float32),
        pltpu.VMEM((_HALF + L,), jnp.float32),
        pltpu.VMEM((_HALF, C), jnp.float32),
        pltpu.VMEM((_HALF, C), jnp.float32),
        pltpu.VMEM((_HALF, C), jnp.float32),
        pltpu.SemaphoreType.DMA,
        pltpu.SemaphoreType.DMA,
    ],
    compiler_params=_sc_params,
)
def _combine(ys_hbm, pos_hbm, wgt_hbm, out_hbm, p0v, p1v, w0v, w1v, buf0,
             buf1, obuf, sem0, sem1):
    tbase = _wid() * TPW
    for h in range(2):
        tb = tbase + h * _HALF
        pltpu.sync_copy(pos_hbm.at[pl.ds(tb, _HALF)], p0v)
        pltpu.sync_copy(pos_hbm.at[pl.ds(T + tb, _HALF)], p1v)
        pltpu.sync_copy(wgt_hbm.at[pl.ds(tb, _HALF)],
                        w0v.at[pl.ds(0, _HALF)])
        pltpu.sync_copy(wgt_hbm.at[pl.ds(T + tb, _HALF)],
                        w1v.at[pl.ds(0, _HALF)])
        c0 = pltpu.async_copy(ys_hbm.at[p0v], buf0, sem0)
        c1 = pltpu.async_copy(ys_hbm.at[p1v], buf1, sem1)
        c0.wait()
        c1.wait()

        def tok(i, _):
            w0 = w0v[pl.ds(i, L)][0]
            w1 = w1v[pl.ds(i, L)][0]
            for k2 in range(C // L):
                s = pl.ds(k2 * L, L)
                obuf[i, s] = w0 * buf0[i, s] + w1 * buf1[i, s]
            return 0

        lax.fori_loop(0, _HALF, tok, 0)
        pltpu.sync_copy(obuf, out_hbm.at[pl.ds(tb, _HALF)])


def kernel(x, w_router, w_v, w_proj):
    x2 = x.reshape(T, C)
    wv_b = w_v.astype(jnp.bfloat16)
    wp_b = w_proj.astype(jnp.bfloat16)
    eidx, wgt, aux = _router(x2, w_router)
    pos, xs, bexp = _route(eidx.reshape(N), x2)
    ys = _ffn(bexp, xs, wv_b, wp_b)
    out2 = _combine(ys, pos, wgt.reshape(N))
    return out2.reshape(B, T, C), aux.reshape(())


# speedup vs baseline: 1.6360x; 1.0495x over previous
import functools

import jax
import jax.numpy as jnp
from jax import lax
from jax.experimental import pallas as pl
from jax.experimental.pallas import tpu as pltpu
from jax.experimental.pallas import tpu_sc as plsc

B, T, C = 1, 2048, 768
E, K, F = 8, 2, 2048
N = T * K
BLK = 256
NB = (N + E * (BLK - 1)) // BLK + 1
CAP = NB * BLK
NC, NS, L = 2, 16, 16
NW = NC * NS
APW = N // NW
TPW = T // NW
NBX = 48

_sc_mesh = plsc.VectorSubcoreMesh(
    core_axis_name="c", subcore_axis_name="s", num_cores=NC, num_subcores=NS)
_sc_params = pltpu.CompilerParams(needs_layout_passes=False)


def _wid():
    return lax.axis_index("s") * NC + lax.axis_index("c")


def _router_body(x_ref, wr_ref, eidx_ref, wgt_ref, aux_ref):
    x = x_ref[...]
    logits = jnp.dot(x, wr_ref[...], preferred_element_type=jnp.float32)
    m = jnp.max(logits, axis=-1, keepdims=True)
    ex = jnp.exp(logits - m)
    probs = ex / jnp.sum(ex, axis=-1, keepdims=True)
    i0 = jnp.argmax(probs, axis=-1).astype(jnp.int32)
    p0 = jnp.max(probs, axis=-1)
    cols = lax.broadcasted_iota(jnp.int32, probs.shape, 1)
    masked = jnp.where(cols == i0[:, None], -jnp.inf, probs)
    i1 = jnp.argmax(masked, axis=-1).astype(jnp.int32)
    p1 = jnp.max(masked, axis=-1)
    den = p0 + p1
    eidx_ref[...] = jnp.concatenate([i0[None, :], i1[None, :]], axis=0)
    wgt_ref[...] = jnp.concatenate([(p0 / den)[None, :], (p1 / den)[None, :]],
                                   axis=0)
    avg = jnp.sum(probs, axis=0, keepdims=True) / T
    aux_ref[...] = E * jnp.sum(avg * avg, keepdims=True)


def _router(x2, w_router):
    return pl.pallas_call(
        _router_body,
        out_shape=[
            jax.ShapeDtypeStruct((K, T), jnp.int32),
            jax.ShapeDtypeStruct((K, T), jnp.float32),
            jax.ShapeDtypeStruct((1, 1), jnp.float32),
        ],
    )(x2, w_router)


@functools.partial(
    pl.kernel,
    out_type=[
        jax.ShapeDtypeStruct((N,), jnp.int32),
        jax.ShapeDtypeStruct((CAP, C), jnp.float32),
        jax.ShapeDtypeStruct((NBX,), jnp.int32),
    ],
    mesh=_sc_mesh,
    scratch_types=[
        pltpu.VMEM((N,), jnp.int32),
        pltpu.VMEM((APW,), jnp.int32),
        pltpu.VMEM((APW, C), jnp.float32),
        pltpu.VMEM((NBX,), jnp.int32),
        pltpu.SemaphoreType.DMA,
    ],
    compiler_params=_sc_params,
)
def _route(eidx_hbm, x_hbm, pos_hbm, xs_hbm, bexp_hbm, ev, posb, rows, bexpb,
           sem):
    wid = _wid()
    base = wid * APW
    t0 = base - jnp.where(wid >= NW // K, T, 0)
    cp = pltpu.async_copy(x_hbm.at[pl.ds(t0, APW)], rows, sem)
    pltpu.sync_copy(eidx_hbm, ev)
    my_vreg = wid * (APW // L)

    def hist_step(i, carry):
        cntv = carry[:E]
        snapv = carry[E:]
        snapv = tuple(
            jnp.where(i == my_vreg, cntv[e], snapv[e]) for e in range(E))
        v = ev[pl.ds(i * L, L)]
        cntv = tuple(
            cntv[e] + jnp.where(v == e, 1, 0).astype(jnp.int32)
            for e in range(E))
        return cntv + snapv

    zv = jnp.zeros((L,), jnp.int32)
    carry = lax.fori_loop(0, N // L, hist_step, (zv,) * (2 * E))
    tot = [jnp.sum(carry[e]) for e in range(E)]
    pre = [jnp.sum(carry[E + e]) for e in range(E)]

    run = jnp.int32(0)
    start, sb = [], []
    for e in range(E):
        start.append(run)
        sb.append(run // BLK)
        padded = ((tot[e] + BLK - 1) // BLK) * BLK
        run = run + padded
    nb_used = run // BLK
    mybase = [start[e] + pre[e] for e in range(E)]

    for r in range(APW // L):
        v = ev[pl.ds(base + r * L, L)]
        posv = jnp.zeros((L,), jnp.int32)
        for e in range(E):
            m = v == e
            mi = jnp.where(m, 1, 0).astype(jnp.int32)
            c = plsc.cumsum(mi)
            posv = jnp.where(m, mybase[e] + c - 1, posv)
            mybase[e] = mybase[e] + jnp.sum(mi)
        posb[pl.ds(r * L, L)] = posv

    pltpu.sync_copy(posb, pos_hbm.at[pl.ds(base, APW)])
    cp.wait()
    pltpu.async_copy(rows, xs_hbm.at[posb], sem).wait()

    @pl.when(wid == 0)
    def _():
        for ch in range(NBX // L):
            bv = jnp.int32(ch * L) + lax.iota(jnp.int32, L)
            val = jnp.zeros((L,), jnp.int32)
            for e in range(1, E):
                val = val + jnp.where(bv >= sb[e], 1, 0).astype(jnp.int32)
            val = jnp.where(bv == NB, nb_used, val)
            bexpb[pl.ds(ch * L, L)] = val
        pltpu.sync_copy(bexpb, bexp_hbm)


def _ffn_body(bexp_ref, xs_ref, wv_ref, wp_ref, ys_ref):
    b = pl.program_id(0)

    @pl.when(b < bexp_ref[NB])
    def _():
        xb = xs_ref[...].astype(jnp.bfloat16)
        gv = jnp.dot(xb, wv_ref[0], preferred_element_type=jnp.float32)
        g = gv[:, :F]
        v = gv[:, F:]
        h = (g * lax.logistic(g) * v).astype(jnp.bfloat16)
        ys_ref[...] = jnp.dot(h, wp_ref[0], preferred_element_type=jnp.float32)


def _ffn(bexp, xs_bf, wv_b, wp_b):
    grid_spec = pltpu.PrefetchScalarGridSpec(
        num_scalar_prefetch=1,
        grid=(NB,),
        in_specs=[
            pl.BlockSpec((BLK, C), lambda b, bexp: (b, 0)),
            pl.BlockSpec((1, C, 2 * F), lambda b, bexp: (bexp[b], 0, 0)),
            pl.BlockSpec((1, F, C), lambda b, bexp: (bexp[b], 0, 0)),
        ],
        out_specs=pl.BlockSpec((BLK, C), lambda b, bexp: (b, 0)),
    )
    return pl.pallas_call(
        _ffn_body,
        grid_spec=grid_spec,
        out_shape=jax.ShapeDtypeStruct((CAP, C), jnp.float32),
        compiler_params=pltpu.CompilerParams(
            dimension_semantics=("arbitrary",)),
    )(bexp, xs_bf, wv_b, wp_b)


_HALF = TPW // 2


@functools.partial(
    pl.kernel,
    out_type=jax.ShapeDtypeStruct((T, C), jnp.float32),
    mesh=_sc_mesh,
    scratch_types=[
        pltpu.VMEM((_HALF,), jnp.int32),
        pltpu.VMEM((_HALF,), jnp.int32),
        pltpu.VMEM((_HALF + L,), jnp.float32),
        pltpu.VMEM((_HALF + L,), jnp.float32),
        pltpu.VMEM((_HALF, C), jnp.float32),
        pltpu.VMEM((_HALF, C), jnp.float32),
        pltpu.VMEM((_HALF, C), jnp.float32),
        pltpu.SemaphoreType.DMA,
        pltpu.SemaphoreType.DMA,
    ],
    compiler_params=_sc_params,
)
def _combine(ys_hbm, pos_hbm, wgt_hbm, out_hbm, p0v, p1v, w0v, w1v, buf0,
             buf1, obuf, sem0, sem1):
    tbase = _wid() * TPW
    for h in range(2):
        tb = tbase + h * _HALF
        pltpu.sync_copy(pos_hbm.at[pl.ds(tb, _HALF)], p0v)
        pltpu.sync_copy(pos_hbm.at[pl.ds(T + tb, _HALF)], p1v)
        pltpu.sync_copy(wgt_hbm.at[pl.ds(tb, _HALF)],
                        w0v.at[pl.ds(0, _HALF)])
        pltpu.sync_copy(wgt_hbm.at[pl.ds(T + tb, _HALF)],
                        w1v.at[pl.ds(0, _HALF)])
        c0 = pltpu.async_copy(ys_hbm.at[p0v], buf0, sem0)
        c1 = pltpu.async_copy(ys_hbm.at[p1v], buf1, sem1)
        c0.wait()
        c1.wait()

        def tok(i, _):
            w0 = w0v[pl.ds(i, L)][0]
            w1 = w1v[pl.ds(i, L)][0]
            for k2 in range(C // L):
                s = pl.ds(k2 * L, L)
                obuf[i, s] = w0 * buf0[i, s] + w1 * buf1[i, s]
            return 0

        lax.fori_loop(0, _HALF, tok, 0)
        pltpu.sync_copy(obuf, out_hbm.at[pl.ds(tb, _HALF)])


def kernel(x, w_router, w_v, w_proj):
    x2 = x.reshape(T, C)
    wv_b = w_v.astype(jnp.bfloat16)
    wp_b = w_proj.astype(jnp.bfloat16)
    eidx, wgt, aux = _router(x2, w_router)
    pos, xs, bexp = _route(eidx.reshape(N), x2)
    ys = _ffn(bexp, xs, wv_b, wp_b)
    out2 = _combine(ys, pos, wgt.reshape(N))
    return out2.reshape(B, T, C), aux.reshape(())


# speedup vs baseline: 2.1306x; 1.3023x over previous
import functools

import jax
import jax.numpy as jnp
from jax import lax
from jax.experimental import pallas as pl
from jax.experimental.pallas import tpu as pltpu
from jax.experimental.pallas import tpu_sc as plsc

B, T, C = 1, 2048, 768
E, K, F = 8, 2, 2048
N = T * K
BLK = 256
NB = (N + E * (BLK - 1)) // BLK + 1
CAP = NB * BLK
NC, NS, L = 2, 16, 16
NW = NC * NS
APW = N // NW
TPW = T // NW
NBX = 48

_sc_mesh = plsc.VectorSubcoreMesh(
    core_axis_name="c", subcore_axis_name="s", num_cores=NC, num_subcores=NS)
_sc_params = pltpu.CompilerParams(needs_layout_passes=False)


def _wid():
    return lax.axis_index("s") * NC + lax.axis_index("c")


def _router_body(x_ref, wr_ref, eidx_ref, wgt_ref, aux_ref):
    x = x_ref[...]
    logits = jnp.dot(x, wr_ref[...], preferred_element_type=jnp.float32)
    m = jnp.max(logits, axis=-1, keepdims=True)
    ex = jnp.exp(logits - m)
    probs = ex / jnp.sum(ex, axis=-1, keepdims=True)
    i0 = jnp.argmax(probs, axis=-1).astype(jnp.int32)
    p0 = jnp.max(probs, axis=-1)
    cols = lax.broadcasted_iota(jnp.int32, probs.shape, 1)
    masked = jnp.where(cols == i0[:, None], -jnp.inf, probs)
    i1 = jnp.argmax(masked, axis=-1).astype(jnp.int32)
    p1 = jnp.max(masked, axis=-1)
    den = p0 + p1
    eidx_ref[...] = jnp.concatenate([i0[None, :], i1[None, :]], axis=0)
    wgt_ref[...] = jnp.concatenate([(p0 / den)[None, :], (p1 / den)[None, :]],
                                   axis=0)
    avg = jnp.sum(probs, axis=0, keepdims=True) / T
    aux_ref[...] = E * jnp.sum(avg * avg, keepdims=True)


def _router(x2, w_router):
    return pl.pallas_call(
        _router_body,
        out_shape=[
            jax.ShapeDtypeStruct((K, T), jnp.int32),
            jax.ShapeDtypeStruct((K, T), jnp.float32),
            jax.ShapeDtypeStruct((1, 1), jnp.float32),
        ],
    )(x2, w_router)


@functools.partial(
    pl.kernel,
    out_type=[
        jax.ShapeDtypeStruct((N,), jnp.int32),
        jax.ShapeDtypeStruct((CAP, C), jnp.float32),
        jax.ShapeDtypeStruct((NBX,), jnp.int32),
    ],
    mesh=_sc_mesh,
    scratch_types=[
        pltpu.VMEM((N,), jnp.int32),
        pltpu.VMEM((APW,), jnp.int32),
        pltpu.VMEM((APW, C), jnp.float32),
        pltpu.VMEM((NBX,), jnp.int32),
        pltpu.SemaphoreType.DMA,
    ],
    compiler_params=_sc_params,
)
def _route(eidx_hbm, x_hbm, pos_hbm, xs_hbm, bexp_hbm, ev, posb, rows, bexpb,
           sem):
    wid = _wid()
    base = wid * APW
    t0 = base - jnp.where(wid >= NW // K, T, 0)
    cp = pltpu.async_copy(x_hbm.at[pl.ds(t0, APW)], rows, sem)
    pltpu.sync_copy(eidx_hbm, ev)
    my_vreg = wid * (APW // L)

    def hist_step(i, carry):
        cntv = carry[:E]
        snapv = carry[E:]
        snapv = tuple(
            jnp.where(i == my_vreg, cntv[e], snapv[e]) for e in range(E))
        v = ev[pl.ds(i * L, L)]
        cntv = tuple(
            cntv[e] + jnp.where(v == e, 1, 0).astype(jnp.int32)
            for e in range(E))
        return cntv + snapv

    zv = jnp.zeros((L,), jnp.int32)
    carry = lax.fori_loop(0, N // L, hist_step, (zv,) * (2 * E))
    tot = [jnp.sum(carry[e]) for e in range(E)]
    pre = [jnp.sum(carry[E + e]) for e in range(E)]

    run = jnp.int32(0)
    start, sb = [], []
    for e in range(E):
        start.append(run)
        sb.append(run // BLK)
        padded = ((tot[e] + BLK - 1) // BLK) * BLK
        run = run + padded
    nb_used = run // BLK
    mybase = [start[e] + pre[e] for e in range(E)]

    for r in range(APW // L):
        v = ev[pl.ds(base + r * L, L)]
        posv = jnp.zeros((L,), jnp.int32)
        for e in range(E):
            m = v == e
            mi = jnp.where(m, 1, 0).astype(jnp.int32)
            c = plsc.cumsum(mi)
            posv = jnp.where(m, mybase[e] + c - 1, posv)
            mybase[e] = mybase[e] + jnp.sum(mi)
        posb[pl.ds(r * L, L)] = posv

    pltpu.sync_copy(posb, pos_hbm.at[pl.ds(base, APW)])
    cp.wait()
    pltpu.async_copy(rows, xs_hbm.at[posb], sem).wait()

    @pl.when(wid == 0)
    def _():
        for ch in range(NBX // L):
            bv = jnp.int32(ch * L) + lax.iota(jnp.int32, L)
            val = jnp.zeros((L,), jnp.int32)
            for e in range(1, E):
                val = val + jnp.where(bv >= sb[e], 1, 0).astype(jnp.int32)
            val = jnp.where(bv == NB, nb_used, val)
            bexpb[pl.ds(ch * L, L)] = val
        pltpu.sync_copy(bexpb, bexp_hbm)


def _ffn_body(bexp_ref, xs_ref, wv_ref, wp_ref, ys_ref):
    b = pl.program_id(0)

    @pl.when(b < bexp_ref[NB])
    def _():
        xb = xs_ref[...]
        gv = jnp.dot(xb, wv_ref[0], preferred_element_type=jnp.float32)
        g = gv[:, :F]
        v = gv[:, F:]
        h = g * lax.logistic(g) * v
        ys_ref[...] = jnp.dot(h, wp_ref[0], preferred_element_type=jnp.float32)


def _ffn(bexp, xs_bf, wv_b, wp_b):
    grid_spec = pltpu.PrefetchScalarGridSpec(
        num_scalar_prefetch=1,
        grid=(NB,),
        in_specs=[
            pl.BlockSpec((BLK, C), lambda b, bexp: (b, 0)),
            pl.BlockSpec((1, C, 2 * F), lambda b, bexp: (bexp[b], 0, 0)),
            pl.BlockSpec((1, F, C), lambda b, bexp: (bexp[b], 0, 0)),
        ],
        out_specs=pl.BlockSpec((BLK, C), lambda b, bexp: (b, 0)),
    )
    return pl.pallas_call(
        _ffn_body,
        grid_spec=grid_spec,
        out_shape=jax.ShapeDtypeStruct((CAP, C), jnp.float32),
        compiler_params=pltpu.CompilerParams(
            dimension_semantics=("arbitrary",),
            vmem_limit_bytes=100 * 1024 * 1024),
    )(bexp, xs_bf, wv_b, wp_b)


_HALF = TPW // 2


@functools.partial(
    pl.kernel,
    out_type=jax.ShapeDtypeStruct((T, C), jnp.float32),
    mesh=_sc_mesh,
    scratch_types=[
        pltpu.VMEM((_HALF,), jnp.int32),
        pltpu.VMEM((_HALF,), jnp.int32),
        pltpu.VMEM((_HALF + L,), jnp.float32),
        pltpu.VMEM((_HALF + L,), jnp.float32),
        pltpu.VMEM((_HALF, C), jnp.float32),
        pltpu.VMEM((_HALF, C), jnp.float32),
        pltpu.VMEM((_HALF, C), jnp.float32),
        pltpu.SemaphoreType.DMA,
        pltpu.SemaphoreType.DMA,
    ],
    compiler_params=_sc_params,
)
def _combine(ys_hbm, pos_hbm, wgt_hbm, out_hbm, p0v, p1v, w0v, w1v, buf0,
             buf1, obuf, sem0, sem1):
    tbase = _wid() * TPW
    for h in range(2):
        tb = tbase + h * _HALF
        pltpu.sync_copy(pos_hbm.at[pl.ds(tb, _HALF)], p0v)
        pltpu.sync_copy(pos_hbm.at[pl.ds(T + tb, _HALF)], p1v)
        pltpu.sync_copy(wgt_hbm.at[pl.ds(tb, _HALF)],
                        w0v.at[pl.ds(0, _HALF)])
        pltpu.sync_copy(wgt_hbm.at[pl.ds(T + tb, _HALF)],
                        w1v.at[pl.ds(0, _HALF)])
        c0 = pltpu.async_copy(ys_hbm.at[p0v], buf0, sem0)
        c1 = pltpu.async_copy(ys_hbm.at[p1v], buf1, sem1)
        c0.wait()
        c1.wait()

        def tok(i, _):
            w0 = w0v[pl.ds(i, L)][0]
            w1 = w1v[pl.ds(i, L)][0]
            for k2 in range(C // L):
                s = pl.ds(k2 * L, L)
                obuf[i, s] = w0 * buf0[i, s] + w1 * buf1[i, s]
            return 0

        lax.fori_loop(0, _HALF, tok, 0)
        pltpu.sync_copy(obuf, out_hbm.at[pl.ds(tb, _HALF)])


def kernel(x, w_router, w_v, w_proj):
    x2 = x.reshape(T, C)
    eidx, wgt, aux = _router(x2, w_router)
    pos, xs, bexp = _route(eidx.reshape(N), x2)
    ys = _ffn(bexp, xs, w_v, w_proj)
    out2 = _combine(ys, pos, wgt.reshape(N))
    return out2.reshape(B, T, C), aux.reshape(())


# speedup vs baseline: 2.1741x; 1.0204x over previous
import functools

import jax
import jax.numpy as jnp
from jax import lax
from jax.experimental import pallas as pl
from jax.experimental.pallas import tpu as pltpu
from jax.experimental.pallas import tpu_sc as plsc

B, T, C = 1, 2048, 768
E, K, F = 8, 2, 2048
N = T * K
BLK = 256
NB = (N + E * (BLK - 1)) // BLK + 1
CAP = NB * BLK
NC, NS, L = 2, 16, 16
NW = NC * NS
APW = N // NW
TPW = T // NW
NBX = 48

_sc_mesh = plsc.VectorSubcoreMesh(
    core_axis_name="c", subcore_axis_name="s", num_cores=NC, num_subcores=NS)
_sc_params = pltpu.CompilerParams(needs_layout_passes=False)


def _wid():
    return lax.axis_index("s") * NC + lax.axis_index("c")


def _router_body(x_ref, wr_ref, eidx_ref, wgt_ref, aux_ref):
    x = x_ref[...]
    logits = jnp.dot(x, wr_ref[...], preferred_element_type=jnp.float32)
    m = jnp.max(logits, axis=-1, keepdims=True)
    ex = jnp.exp(logits - m)
    probs = ex / jnp.sum(ex, axis=-1, keepdims=True)
    i0 = jnp.argmax(probs, axis=-1).astype(jnp.int32)
    p0 = jnp.max(probs, axis=-1)
    cols = lax.broadcasted_iota(jnp.int32, probs.shape, 1)
    masked = jnp.where(cols == i0[:, None], -jnp.inf, probs)
    i1 = jnp.argmax(masked, axis=-1).astype(jnp.int32)
    p1 = jnp.max(masked, axis=-1)
    den = p0 + p1
    eidx_ref[...] = jnp.concatenate([i0[None, :], i1[None, :]], axis=0)
    wgt_ref[...] = jnp.concatenate([(p0 / den)[None, :], (p1 / den)[None, :]],
                                   axis=0)
    avg = jnp.sum(probs, axis=0, keepdims=True) / T
    aux_ref[...] = E * jnp.sum(avg * avg, keepdims=True)


def _router(x2, w_router):
    return pl.pallas_call(
        _router_body,
        out_shape=[
            jax.ShapeDtypeStruct((K, T), jnp.int32),
            jax.ShapeDtypeStruct((K, T), jnp.float32),
            jax.ShapeDtypeStruct((1, 1), jnp.float32),
        ],
    )(x2, w_router)


_TVR = T // L


@functools.partial(
    pl.kernel,
    out_type=[
        jax.ShapeDtypeStruct((K, T), jnp.int32),
        jax.ShapeDtypeStruct((CAP, C), jnp.float32),
        jax.ShapeDtypeStruct((NBX,), jnp.int32),
    ],
    mesh=_sc_mesh,
    scratch_types=[
        pltpu.VMEM((K, T), jnp.int32),
        pltpu.VMEM((TPW,), jnp.int32),
        pltpu.VMEM((TPW,), jnp.int32),
        pltpu.VMEM((TPW, C), jnp.float32),
        pltpu.VMEM((NBX,), jnp.int32),
        pltpu.SemaphoreType.DMA,
        pltpu.SemaphoreType.DMA,
    ],
    compiler_params=_sc_params,
)
def _route(eidx_hbm, x_hbm, pos_hbm, xs_hbm, bexp_hbm, ev, posba, posbb, rows,
           bexpb, sema, semb):
    wid = _wid()
    tw = wid * TPW
    cp = pltpu.async_copy(x_hbm.at[pl.ds(tw, TPW)], rows, sema)
    pltpu.sync_copy(eidx_hbm, ev)
    my_vreg = wid * (TPW // L)

    def make_hist(k):
        def hist_step(i, carry):
            cntv = carry[:E]
            snapv = carry[E:]
            snapv = tuple(
                jnp.where(i == my_vreg, cntv[e], snapv[e]) for e in range(E))
            v = ev[k, pl.ds(i * L, L)]
            cntv = tuple(
                cntv[e] + jnp.where(v == e, 1, 0).astype(jnp.int32)
                for e in range(E))
            return cntv + snapv
        return hist_step

    zv = jnp.zeros((L,), jnp.int32)
    carry = lax.fori_loop(0, _TVR, make_hist(0), (zv,) * (2 * E))
    snap_a = carry[E:]
    carry = lax.fori_loop(0, _TVR, make_hist(1), carry[:E] + (zv,) * E)
    tot = [jnp.sum(carry[e]) for e in range(E)]
    pre_a = [jnp.sum(snap_a[e]) for e in range(E)]
    pre_b = [jnp.sum(carry[E + e]) for e in range(E)]

    run = jnp.int32(0)
    start, sb = [], []
    for e in range(E):
        start.append(run)
        sb.append(run // BLK)
        padded = ((tot[e] + BLK - 1) // BLK) * BLK
        run = run + padded
    nb_used = run // BLK

    def pos_pass(k, mybase, posb):
        for r in range(TPW // L):
            v = ev[k, pl.ds(tw + r * L, L)]
            posv = jnp.zeros((L,), jnp.int32)
            for e in range(E):
                m = v == e
                mi = jnp.where(m, 1, 0).astype(jnp.int32)
                c = plsc.cumsum(mi)
                posv = jnp.where(m, mybase[e] + c - 1, posv)
                mybase[e] = mybase[e] + jnp.sum(mi)
            posb[pl.ds(r * L, L)] = posv

    pos_pass(0, [start[e] + pre_a[e] for e in range(E)], posba)
    pos_pass(1, [start[e] + pre_b[e] for e in range(E)], posbb)

    pltpu.sync_copy(posba, pos_hbm.at[0, pl.ds(tw, TPW)])
    pltpu.sync_copy(posbb, pos_hbm.at[1, pl.ds(tw, TPW)])
    cp.wait()
    ca = pltpu.async_copy(rows, xs_hbm.at[posba], sema)
    cb = pltpu.async_copy(rows, xs_hbm.at[posbb], semb)
    ca.wait()
    cb.wait()

    @pl.when(wid == 0)
    def _():
        for ch in range(NBX // L):
            bv = jnp.int32(ch * L) + lax.iota(jnp.int32, L)
            val = jnp.zeros((L,), jnp.int32)
            for e in range(1, E):
                val = val + jnp.where(bv >= sb[e], 1, 0).astype(jnp.int32)
            val = jnp.where(bv == NB, nb_used, val)
            bexpb[pl.ds(ch * L, L)] = val
        pltpu.sync_copy(bexpb, bexp_hbm)


def _ffn_body(bexp_ref, xs_ref, wv_ref, wp_ref, ys_ref):
    b = pl.program_id(0)

    @pl.when(b < bexp_ref[NB])
    def _():
        xb = xs_ref[...]
        gv = jnp.dot(xb, wv_ref[0], preferred_element_type=jnp.float32)
        g = gv[:, :F]
        v = gv[:, F:]
        h = g * lax.logistic(g) * v
        ys_ref[...] = jnp.dot(h, wp_ref[0], preferred_element_type=jnp.float32)


def _ffn(bexp, xs_bf, wv_b, wp_b):
    grid_spec = pltpu.PrefetchScalarGridSpec(
        num_scalar_prefetch=1,
        grid=(NB,),
        in_specs=[
            pl.BlockSpec((BLK, C), lambda b, bexp: (b, 0)),
            pl.BlockSpec((1, C, 2 * F), lambda b, bexp: (bexp[b], 0, 0)),
            pl.BlockSpec((1, F, C), lambda b, bexp: (bexp[b], 0, 0)),
        ],
        out_specs=pl.BlockSpec((BLK, C), lambda b, bexp: (b, 0)),
    )
    return pl.pallas_call(
        _ffn_body,
        grid_spec=grid_spec,
        out_shape=jax.ShapeDtypeStruct((CAP, C), jnp.float32),
        compiler_params=pltpu.CompilerParams(
            dimension_semantics=("arbitrary",),
            vmem_limit_bytes=100 * 1024 * 1024),
    )(bexp, xs_bf, wv_b, wp_b)


_HALF = TPW // 2


@functools.partial(
    pl.kernel,
    out_type=jax.ShapeDtypeStruct((T, C), jnp.float32),
    mesh=_sc_mesh,
    scratch_types=[
        pltpu.VMEM((_HALF,), jnp.int32),
        pltpu.VMEM((_HALF,), jnp.int32),
        pltpu.VMEM((_HALF + L,), jnp.float32),
        pltpu.VMEM((_HALF + L,), jnp.float32),
        pltpu.VMEM((_HALF, C), jnp.float32),
        pltpu.VMEM((_HALF, C), jnp.float32),
        pltpu.VMEM((_HALF, C), jnp.float32),
        pltpu.SemaphoreType.DMA,
        pltpu.SemaphoreType.DMA,
    ],
    compiler_params=_sc_params,
)
def _combine(ys_hbm, pos_hbm, wgt_hbm, out_hbm, p0v, p1v, w0v, w1v, buf0,
             buf1, obuf, sem0, sem1):
    tbase = _wid() * TPW
    for h in range(2):
        tb = tbase + h * _HALF
        pltpu.sync_copy(pos_hbm.at[0, pl.ds(tb, _HALF)], p0v)
        pltpu.sync_copy(pos_hbm.at[1, pl.ds(tb, _HALF)], p1v)
        pltpu.sync_copy(wgt_hbm.at[0, pl.ds(tb, _HALF)],
                        w0v.at[pl.ds(0, _HALF)])
        pltpu.sync_copy(wgt_hbm.at[1, pl.ds(tb, _HALF)],
                        w1v.at[pl.ds(0, _HALF)])
        c0 = pltpu.async_copy(ys_hbm.at[p0v], buf0, sem0)
        c1 = pltpu.async_copy(ys_hbm.at[p1v], buf1, sem1)
        c0.wait()
        c1.wait()

        def tok(i, _):
            w0 = w0v[pl.ds(i, L)][0]
            w1 = w1v[pl.ds(i, L)][0]
            for k2 in range(C // L):
                s = pl.ds(k2 * L, L)
                obuf[i, s] = w0 * buf0[i, s] + w1 * buf1[i, s]
            return 0

        lax.fori_loop(0, _HALF, tok, 0)
        pltpu.sync_copy(obuf, out_hbm.at[pl.ds(tb, _HALF)])


def kernel(x, w_router, w_v, w_proj):
    x2 = x.reshape(T, C)
    eidx, wgt, aux = _router(x2, w_router)
    pos, xs, bexp = _route(eidx, x2)
    ys = _ffn(bexp, xs, w_v, w_proj)
    out2 = _combine(ys, pos, wgt)
    return out2.reshape(B, T, C), aux.reshape(())
